# baseline (speedup 1.0000x reference)
import functools

import jax
import jax.numpy as jnp
from jax import lax
from jax.experimental import pallas as pl
from jax.experimental.pallas import tpu as pltpu
from jax.experimental.pallas import tpu_sc as plsc

NC = 2
NS = 16
NW = NC * NS
CHUNK = 80
NRB = 3
NIB = 6
UNROLL = 6


def _sc_aggregate(xaug, edges2, zinit, n_pad, da, nb0, nb1):
    rows_per_tile = n_pad // NS
    nbp = nb0 + nb1

    mesh = plsc.VectorSubcoreMesh(core_axis_name="c", subcore_axis_name="s")

    @functools.partial(
        pl.kernel,
        out_type=jax.ShapeDtypeStruct((NC, n_pad, da), jnp.float32),
        mesh=mesh,
        scratch_types=(
            [pltpu.VMEM((CHUNK,), jnp.int32) for _ in range(NIB)]
            + [pltpu.VMEM((CHUNK,), jnp.int32) for _ in range(NIB)]
            + [
                pltpu.VMEM((NRB, CHUNK, da), jnp.float32),
                pltpu.VMEM_SHARED((n_pad, da), jnp.float32),
                pltpu.SemaphoreType.DMA((NIB,)),
                pltpu.SemaphoreType.DMA((NRB,)),
                pltpu.SemaphoreType.DMA((NRB,)),
            ]
        ),
        compiler_params=pltpu.CompilerParams(use_tc_tiling_on_sc=False),
    )
    def agg(xaug_hbm, edges_hbm, zero_hbm, out_hbm, *refs):
        sidx = refs[:NIB]
        didx = refs[NIB:2 * NIB]
        rows_v, acc_sh, isem, gsem, ssem = refs[2 * NIB:]
        cid = lax.axis_index("c")
        sid = lax.axis_index("s")
        base = sid * nbp + cid * nb0
        nb = lax.select(cid == 0, nb0, nb1)
        t0 = sid * rows_per_tile

        def start_idx(k, i):
            pltpu.async_copy(edges_hbm.at[base + k, 0], sidx[i], isem.at[i])
            pltpu.async_copy(edges_hbm.at[base + k, 1], didx[i], isem.at[i])

        def wait_idx(i):
            pltpu.make_async_copy(edges_hbm.at[base, 0], sidx[i],
                                  isem.at[i]).wait()
            pltpu.make_async_copy(edges_hbm.at[base, 1], didx[i],
                                  isem.at[i]).wait()

        def start_gather(i, r):
            pltpu.async_copy(xaug_hbm.at[sidx[i]], rows_v.at[r], gsem.at[r])

        def wait_gather(r):
            pltpu.make_async_copy(xaug_hbm.at[sidx[0]], rows_v.at[r],
                                  gsem.at[r]).wait()

        def start_scatter(i, r):
            pltpu.async_copy(rows_v.at[r], acc_sh.at[didx[i]],
                             ssem.at[r], add=True)

        def wait_scatter(r):
            pltpu.make_async_copy(rows_v.at[r], acc_sh.at[didx[0]],
                                  ssem.at[r]).wait()

        with jax.named_scope("zeroinit"):
            pltpu.sync_copy(zero_hbm.at[pl.ds(t0, rows_per_tile)],
                            acc_sh.at[pl.ds(t0, rows_per_tile)])
            plsc.subcore_barrier()

        sc1 = jax.named_scope("edgeloop")
        sc1.__enter__()
        start_idx(0, 0)
        start_idx(1, 1)
        start_idx(2, 2)
        start_idx(3, 3)
        wait_idx(0)
        start_gather(0, 0)
        wait_idx(1)
        start_gather(1, 1)

        def body(g, carry):
            for b in range(UNROLL):
                k = g * UNROLL + b
                r = b % NRB

                wait_gather(r)
                start_scatter(b % NIB, r)

                @pl.when(k >= 1)
                def _():
                    wait_scatter((r + 2) % NRB)

                @pl.when(k + 2 < nb)
                def _():
                    wait_idx((b + 2) % NIB)
                    start_gather((b + 2) % NIB, (b + 2) % NRB)

                @pl.when(k + 4 < nb)
                def _():
                    start_idx(k + 4, (b + 4) % NIB)
            return carry

        lax.fori_loop(0, nb // UNROLL, body, 0)
        wait_scatter(2)
        plsc.subcore_barrier()
        sc1.__exit__(None, None, None)

        with jax.named_scope("writeout"):
            pltpu.sync_copy(acc_sh.at[pl.ds(t0, rows_per_tile)],
                            out_hbm.at[cid, pl.ds(t0, rows_per_tile)])

    return agg(xaug, edges2, zinit)


def _tc_finish_body(p_ref, x_ref, wl_ref, b_ref, wr_ref, o_ref, *, d):
    p = p_ref[...]
    summed = p[0, :, :d] + p[1, :, :d]
    cnt = p[0, :, d] + p[1, :, d]
    mean = summed / jnp.maximum(cnt, 1.0)[:, None]
    o_ref[...] = (
        jnp.dot(mean, wl_ref[...], preferred_element_type=jnp.float32)
        + b_ref[...]
        + jnp.dot(x_ref[...], wr_ref[...], preferred_element_type=jnp.float32)
    )


def kernel(x, edge_index, W_l, b_l, W_r):
    n, d = x.shape
    h = W_l.shape[1]
    e = edge_index.shape[1]
    da = ((d + 1 + 15) // 16) * 16
    n_pad = ((n + 1 + 127) // 128) * 128

    nbp = -(-e // (NS * CHUNK))
    nbp = -(-nbp // 24) * 24
    nb1 = nbp // 4
    nb0 = nbp - nb1
    e_pad = NS * nbp * CHUNK

    xaug = jnp.pad(
        jnp.concatenate([x, jnp.ones((n, 1), x.dtype)], axis=1),
        ((0, n_pad - n), (0, da - d - 1)),
    )
    src2 = jnp.concatenate(
        [edge_index[0], jnp.zeros((e_pad - e,), jnp.int32)]
    ).reshape(NS * nbp, 1, CHUNK)
    pad_dst = n + jnp.arange(e_pad - e, dtype=jnp.int32) % (n_pad - n)
    dst2 = jnp.concatenate(
        [edge_index[1], pad_dst]
    ).reshape(NS * nbp, 1, CHUNK)
    edges2 = jnp.concatenate([src2, dst2], axis=1)
    zinit = jnp.zeros((n_pad, da), jnp.float32)

    partial = _sc_aggregate(xaug, edges2, zinit, n_pad, da, nb0, nb1)

    blk = 1000
    grid = (n // blk,)
    out = pl.pallas_call(
        functools.partial(_tc_finish_body, d=d),
        grid=grid,
        in_specs=[
            pl.BlockSpec((NC, blk, da), lambda i: (0, i, 0)),
            pl.BlockSpec((blk, d), lambda i: (i, 0)),
            pl.BlockSpec((d, h), lambda i: (0, 0)),
            pl.BlockSpec((1, h), lambda i: (0, 0)),
            pl.BlockSpec((d, h), lambda i: (0, 0)),
        ],
        out_specs=pl.BlockSpec((blk, h), lambda i: (i, 0)),
        out_shape=jax.ShapeDtypeStruct((n, h), jnp.float32),
    )(partial, x, W_l, b_l.reshape(1, h), W_r)
    return out


# speedup vs baseline: 1.6021x; 1.6021x over previous
import functools

import jax
import jax.numpy as jnp
from jax import lax
from jax.experimental import pallas as pl
from jax.experimental.pallas import tpu as pltpu
from jax.experimental.pallas import tpu_sc as plsc

NC = 2
NS = 16
NW = NC * NS
CHUNK = 128
NRB = 2
NIB = 4
L = 16


def _sc_aggregate(x, src1, dst1, zinit, n_pad, d, nb0, nb1):
    rows_per_tile = n_pad // NS
    nbp = nb0 + nb1

    mesh = plsc.VectorSubcoreMesh(core_axis_name="c", subcore_axis_name="s")

    @functools.partial(
        pl.kernel,
        out_type=[
            jax.ShapeDtypeStruct((NC, n_pad, d), jnp.float32),
            jax.ShapeDtypeStruct((NC, NS, n_pad), jnp.float32),
        ],
        mesh=mesh,
        scratch_types=(
            [pltpu.VMEM((CHUNK,), jnp.int32) for _ in range(NIB)]
            + [pltpu.VMEM((CHUNK,), jnp.int32) for _ in range(NIB)]
            + [
                pltpu.VMEM((NRB, CHUNK, d), jnp.float32),
                pltpu.VMEM((n_pad,), jnp.float32),
                pltpu.VMEM_SHARED((n_pad, d), jnp.float32),
                pltpu.SemaphoreType.DMA((NIB,)),
                pltpu.SemaphoreType.DMA((NRB,)),
                pltpu.SemaphoreType.DMA((NRB,)),
            ]
        ),
        compiler_params=pltpu.CompilerParams(use_tc_tiling_on_sc=False,
                                             needs_layout_passes=False),
    )
    def agg(x_hbm, src_hbm, dst_hbm, zero_hbm, out_hbm, cnt_hbm, *refs):
        sidx = refs[:NIB]
        didx = refs[NIB:2 * NIB]
        rows_v, cnt_v, acc_sh, isem, gsem, ssem = refs[2 * NIB:]
        cid = lax.axis_index("c")
        sid = lax.axis_index("s")
        base = sid * nbp + cid * nb0
        nb = lax.select(cid == 0, nb0, nb1)
        t0 = sid * rows_per_tile
        ones = jnp.ones((L,), jnp.float32)

        def start_idx(k, i):
            off = (base + k) * CHUNK
            pltpu.async_copy(src_hbm.at[pl.ds(off, CHUNK)], sidx[i],
                             isem.at[i])
            pltpu.async_copy(dst_hbm.at[pl.ds(off, CHUNK)], didx[i],
                             isem.at[i])

        def wait_idx(i):
            pltpu.make_async_copy(src_hbm.at[pl.ds(0, CHUNK)], sidx[i],
                                  isem.at[i]).wait()
            pltpu.make_async_copy(dst_hbm.at[pl.ds(0, CHUNK)], didx[i],
                                  isem.at[i]).wait()

        def start_gather(i, r):
            pltpu.async_copy(x_hbm.at[sidx[i]], rows_v.at[r], gsem.at[r])

        def wait_gather(r):
            pltpu.make_async_copy(x_hbm.at[sidx[0]], rows_v.at[r],
                                  gsem.at[r]).wait()

        def start_scatter(i, r):
            pltpu.async_copy(rows_v.at[r], acc_sh.at[didx[i]],
                             ssem.at[r], add=True)

        def wait_scatter(r):
            pltpu.make_async_copy(rows_v.at[r], acc_sh.at[didx[0]],
                                  ssem.at[r]).wait()

        pltpu.sync_copy(zero_hbm.at[pl.ds(t0, rows_per_tile)],
                        acc_sh.at[pl.ds(t0, rows_per_tile)])

        def zbody(j, carry):
            cnt_v[pl.ds(j * L, L)] = jnp.zeros((L,), jnp.float32)
            return carry

        lax.fori_loop(0, n_pad // L, zbody, 0)
        plsc.subcore_barrier()

        start_idx(0, 0)
        start_idx(1, 1)
        wait_idx(0)
        start_gather(0, 0)

        def body(g, carry):
            for b in range(4):
                k = g * 4 + b
                r = b % NRB
                i = b % NIB

                @pl.when(k >= 2)
                def _():
                    wait_scatter(r)

                @pl.when(k + 2 < nb)
                def _():
                    start_idx(k + 2, (b + 2) % NIB)

                wait_gather(r)
                start_scatter(i, r)

                for j in range(CHUNK // L):
                    dv = didx[i][pl.ds(j * L, L)]
                    plsc.addupdate_scatter(cnt_v, [dv], ones)

                @pl.when(k + 1 < nb)
                def _():
                    wait_idx((b + 1) % NIB)
                    start_gather((b + 1) % NIB, (b + 1) % NRB)
            return carry

        lax.fori_loop(0, nb // 4, body, 0)
        wait_scatter(0)
        wait_scatter(1)
        plsc.subcore_barrier()

        pltpu.sync_copy(acc_sh.at[pl.ds(t0, rows_per_tile)],
                        out_hbm.at[cid, pl.ds(t0, rows_per_tile)])
        pltpu.sync_copy(cnt_v, cnt_hbm.at[cid, sid])

    return agg(x, src1, dst1, zinit)


def _tc_finish_body(p_ref, c_ref, x_ref, wl_ref, b_ref, wr_ref, o_ref):
    p = p_ref[...]
    summed = p[0] + p[1]
    cnt = jnp.sum(c_ref[...], axis=0)
    mean = summed / jnp.maximum(cnt, 1.0)
    o_ref[...] = (
        jnp.dot(mean, wl_ref[...], preferred_element_type=jnp.float32)
        + b_ref[...]
        + jnp.dot(x_ref[...], wr_ref[...], preferred_element_type=jnp.float32)
    )


def kernel(x, edge_index, W_l, b_l, W_r):
    n, d = x.shape
    h = W_l.shape[1]
    e = edge_index.shape[1]
    n_pad = ((n + 1 + 127) // 128) * 128

    nbp = -(-e // (NS * CHUNK))
    nbp = -(-nbp // 8) * 8
    nb1 = nbp // 4
    nb0 = nbp - nb1
    e_pad = NS * nbp * CHUNK

    src1 = jnp.concatenate([edge_index[0], jnp.zeros((e_pad - e,), jnp.int32)])
    pad_dst = n + jnp.arange(e_pad - e, dtype=jnp.int32) % (n_pad - n)
    dst1 = jnp.concatenate([edge_index[1], pad_dst])
    zinit = jnp.zeros((n_pad, d), jnp.float32)

    partial, cnts = _sc_aggregate(x, src1, dst1, zinit, n_pad, d, nb0, nb1)
    cnts3 = cnts.reshape(NW, n_pad, 1)

    blk = 1000
    grid = (n // blk,)
    out = pl.pallas_call(
        _tc_finish_body,
        grid=grid,
        in_specs=[
            pl.BlockSpec((NC, blk, d), lambda i: (0, i, 0)),
            pl.BlockSpec((NW, blk, 1), lambda i: (0, i, 0)),
            pl.BlockSpec((blk, d), lambda i: (i, 0)),
            pl.BlockSpec((d, h), lambda i: (0, 0)),
            pl.BlockSpec((1, h), lambda i: (0, 0)),
            pl.BlockSpec((d, h), lambda i: (0, 0)),
        ],
        out_specs=pl.BlockSpec((blk, h), lambda i: (i, 0)),
        out_shape=jax.ShapeDtypeStruct((n, h), jnp.float32),
    )(partial, cnts3, x, W_l, b_l.reshape(1, h), W_r)
    return out


# speedup vs baseline: 2.1437x; 1.3380x over previous
import functools

import jax
import jax.numpy as jnp
from jax import lax
from jax.experimental import pallas as pl
from jax.experimental.pallas import tpu as pltpu
from jax.experimental.pallas import tpu_sc as plsc

NC = 2
NS = 16
NW = NC * NS
CHUNK = 128
NRB = 2
NIB = 4


def _sc_aggregate(xaug, src1, dst1, zinit, n_pad, da, nb0, nb1):
    rows_per_tile = n_pad // NS
    nbp = nb0 + nb1

    mesh = plsc.VectorSubcoreMesh(core_axis_name="c", subcore_axis_name="s")

    @functools.partial(
        pl.kernel,
        out_type=jax.ShapeDtypeStruct((NC, n_pad, da), jnp.float32),
        mesh=mesh,
        scratch_types=[
            pltpu.VMEM((NIB, 2, CHUNK), jnp.int32),
            pltpu.VMEM((NRB, CHUNK, da), jnp.float32),
            pltpu.VMEM_SHARED((n_pad, da), jnp.float32),
            pltpu.SemaphoreType.DMA((NIB,)),
            pltpu.SemaphoreType.DMA((NRB,)),
            pltpu.SemaphoreType.DMA((NRB,)),
        ],
        compiler_params=pltpu.CompilerParams(use_tc_tiling_on_sc=False),
    )
    def agg(xaug_hbm, src_hbm, dst_hbm, zero_hbm, out_hbm,
            idx_v, rows_v, acc_sh, isem, gsem, ssem):
        cid = lax.axis_index("c")
        sid = lax.axis_index("s")
        base = sid * nbp + cid * nb0
        nb = lax.select(cid == 0, nb0, nb1)
        t0 = sid * rows_per_tile

        def start_idx(k, i):
            off = (base + k) * CHUNK
            pltpu.async_copy(src_hbm.at[pl.ds(off, CHUNK)], idx_v.at[i, 0],
                             isem.at[i])
            pltpu.async_copy(dst_hbm.at[pl.ds(off, CHUNK)], idx_v.at[i, 1],
                             isem.at[i])

        def wait_idx(i):
            pltpu.make_async_copy(src_hbm.at[pl.ds(0, CHUNK)], idx_v.at[i, 0],
                                  isem.at[i]).wait()
            pltpu.make_async_copy(dst_hbm.at[pl.ds(0, CHUNK)], idx_v.at[i, 1],
                                  isem.at[i]).wait()

        def start_gather(i, r):
            pltpu.async_copy(xaug_hbm.at[idx_v.at[i, 0]], rows_v.at[r],
                             gsem.at[r])

        def wait_gather(r):
            pltpu.make_async_copy(xaug_hbm.at[idx_v.at[0, 0]], rows_v.at[r],
                                  gsem.at[r]).wait()

        def start_scatter(i, r):
            pltpu.async_copy(rows_v.at[r], acc_sh.at[idx_v.at[i, 1]],
                             ssem.at[r], add=True)

        def wait_scatter(r):
            pltpu.make_async_copy(rows_v.at[r], acc_sh.at[idx_v.at[0, 1]],
                                  ssem.at[r]).wait()

        with jax.named_scope("zeroinit"):
            pltpu.sync_copy(zero_hbm.at[pl.ds(t0, rows_per_tile)],
                            acc_sh.at[pl.ds(t0, rows_per_tile)])
            plsc.subcore_barrier()

        sc1 = jax.named_scope("edgeloop")
        sc1.__enter__()
        start_idx(0, 0)
        start_idx(1, 1)
        wait_idx(0)
        start_gather(0, 0)

        def body(g, carry):
            for b in range(4):
                k = g * 4 + b
                r = b % NRB
                i = b % NIB

                @pl.when(k >= 2)
                def _():
                    wait_scatter(r)

                @pl.when(k + 2 < nb)
                def _():
                    start_idx(k + 2, (b + 2) % NIB)

                wait_gather(r)
                start_scatter(i, r)

                @pl.when(k + 1 < nb)
                def _():
                    wait_idx((b + 1) % NIB)
                    start_gather((b + 1) % NIB, (b + 1) % NRB)
            return carry

        lax.fori_loop(0, nb // 4, body, 0)
        wait_scatter(0)
        wait_scatter(1)
        plsc.subcore_barrier()
        sc1.__exit__(None, None, None)

        with jax.named_scope("writeout"):
            pltpu.sync_copy(acc_sh.at[pl.ds(t0, rows_per_tile)],
                            out_hbm.at[cid, pl.ds(t0, rows_per_tile)])

    return agg(xaug, src1, dst1, zinit)


def _tc_finish_body(p_ref, x_ref, wl_ref, b_ref, wr_ref, o_ref, *, d):
    p = p_ref[...]
    summed = p[0, :, :d] + p[1, :, :d]
    cnt = p[0, :, d] + p[1, :, d]
    mean = summed / jnp.maximum(cnt, 1.0)[:, None]
    o_ref[...] = (
        jnp.dot(mean, wl_ref[...], preferred_element_type=jnp.float32)
        + b_ref[...]
        + jnp.dot(x_ref[...], wr_ref[...], preferred_element_type=jnp.float32)
    )


def kernel(x, edge_index, W_l, b_l, W_r):
    n, d = x.shape
    h = W_l.shape[1]
    e = edge_index.shape[1]
    da = ((d + 1 + 15) // 16) * 16
    n_pad = ((n + 1 + 127) // 128) * 128

    nbp = -(-e // (NS * CHUNK))
    nbp = -(-nbp // 8) * 8
    nb1 = nbp // 4
    nb0 = nbp - nb1
    e_pad = NS * nbp * CHUNK

    xaug = jnp.pad(
        jnp.concatenate([x, jnp.ones((n, 1), x.dtype)], axis=1),
        ((0, n_pad - n), (0, da - d - 1)),
    )
    src1 = jnp.concatenate([edge_index[0], jnp.zeros((e_pad - e,), jnp.int32)])
    pad_dst = n + jnp.arange(e_pad - e, dtype=jnp.int32) % (n_pad - n)
    dst1 = jnp.concatenate([edge_index[1], pad_dst])
    zinit = jnp.zeros((n_pad, da), jnp.float32)

    partial = _sc_aggregate(xaug, src1, dst1, zinit, n_pad, da, nb0, nb1)

    blk = 1000
    grid = (n // blk,)
    out = pl.pallas_call(
        functools.partial(_tc_finish_body, d=d),
        grid=grid,
        in_specs=[
            pl.BlockSpec((NC, blk, da), lambda i: (0, i, 0)),
            pl.BlockSpec((blk, d), lambda i: (i, 0)),
            pl.BlockSpec((d, h), lambda i: (0, 0)),
            pl.BlockSpec((1, h), lambda i: (0, 0)),
            pl.BlockSpec((d, h), lambda i: (0, 0)),
        ],
        out_specs=pl.BlockSpec((blk, h), lambda i: (i, 0)),
        out_shape=jax.ShapeDtypeStruct((n, h), jnp.float32),
    )(partial, x, W_l, b_l.reshape(1, h), W_r)
    return out


# speedup vs baseline: 2.5272x; 1.1789x over previous
import functools

import jax
import jax.numpy as jnp
from jax import lax
from jax.experimental import pallas as pl
from jax.experimental.pallas import tpu as pltpu
from jax.experimental.pallas import tpu_sc as plsc

NC = 2
NS = 16
NW = NC * NS
CHUNK = 80


def _sc_aggregate(xaug, src, dst, zinit, n_pad, da):
    e = src.shape[0]
    epw = e // NW
    nchunk = epw // CHUNK
    rows_per_tile = n_pad // NS

    mesh = plsc.VectorSubcoreMesh(core_axis_name="c", subcore_axis_name="s")

    @functools.partial(
        pl.kernel,
        out_type=jax.ShapeDtypeStruct((NC, n_pad, da), jnp.float32),
        mesh=mesh,
        scratch_types=[
            pltpu.VMEM((CHUNK,), jnp.int32),
            pltpu.VMEM((CHUNK,), jnp.int32),
            pltpu.VMEM((CHUNK, da), jnp.float32),
            pltpu.VMEM_SHARED((n_pad, da), jnp.float32),
            pltpu.SemaphoreType.DMA,
        ],
        compiler_params=pltpu.CompilerParams(use_tc_tiling_on_sc=False),
    )
    def agg(xaug_hbm, src_hbm, dst_hbm, zero_hbm, out_hbm,
            sidx_v, didx_v, rows_v, acc_sh, sem):
        cid = lax.axis_index("c")
        sid = lax.axis_index("s")
        wid = sid * NC + cid
        base = wid * epw
        t0 = sid * rows_per_tile

        pltpu.sync_copy(zero_hbm.at[pl.ds(t0, rows_per_tile)],
                        acc_sh.at[pl.ds(t0, rows_per_tile)])
        plsc.subcore_barrier()

        def body(j, carry):
            off = base + j * CHUNK
            pltpu.sync_copy(src_hbm.at[pl.ds(off, CHUNK)], sidx_v)
            pltpu.sync_copy(dst_hbm.at[pl.ds(off, CHUNK)], didx_v)
            pltpu.async_copy(xaug_hbm.at[sidx_v], rows_v, sem).wait()
            pltpu.sync_copy(rows_v, acc_sh.at[didx_v], add=True)
            return carry

        lax.fori_loop(0, nchunk, body, 0)
        plsc.subcore_barrier()

        pltpu.sync_copy(acc_sh.at[pl.ds(t0, rows_per_tile)],
                        out_hbm.at[cid, pl.ds(t0, rows_per_tile)])

    return agg(xaug, src, dst, zinit)


def _tc_finish_body(p_ref, x_ref, wl_ref, b_ref, wr_ref, o_ref, *, d):
    p = p_ref[...]
    summed = p[0, :, :d] + p[1, :, :d]
    cnt = p[0, :, d] + p[1, :, d]
    mean = summed / jnp.maximum(cnt, 1.0)[:, None]
    o_ref[...] = (
        jnp.dot(mean, wl_ref[...], preferred_element_type=jnp.float32)
        + b_ref[...]
        + jnp.dot(x_ref[...], wr_ref[...], preferred_element_type=jnp.float32)
    )


def kernel(x, edge_index, W_l, b_l, W_r):
    n, d = x.shape
    h = W_l.shape[1]
    e = edge_index.shape[1]
    da = ((d + 1 + 15) // 16) * 16
    n_pad = ((n + 8 * NW - 1) // (8 * NW)) * (8 * NW)
    assert e % (NW * CHUNK) == 0

    xaug = jnp.pad(
        jnp.concatenate([x, jnp.ones((n, 1), x.dtype)], axis=1),
        ((0, n_pad - n), (0, da - d - 1)),
    )
    src = edge_index[0]
    dst = edge_index[1]
    zinit = jnp.zeros((n_pad, da), jnp.float32)

    partial = _sc_aggregate(xaug, src, dst, zinit, n_pad, da)

    blk = 1000
    grid = (n // blk,)
    out = pl.pallas_call(
        functools.partial(_tc_finish_body, d=d),
        grid=grid,
        in_specs=[
            pl.BlockSpec((NC, blk, da), lambda i: (0, i, 0)),
            pl.BlockSpec((blk, d), lambda i: (i, 0)),
            pl.BlockSpec((d, h), lambda i: (0, 0)),
            pl.BlockSpec((1, h), lambda i: (0, 0)),
            pl.BlockSpec((d, h), lambda i: (0, 0)),
        ],
        out_specs=pl.BlockSpec((blk, h), lambda i: (i, 0)),
        out_shape=jax.ShapeDtypeStruct((n, h), jnp.float32),
    )(partial, x, W_l, b_l.reshape(1, h), W_r)
    return out


# speedup vs baseline: 2.9305x; 1.1596x over previous
import functools

import jax
import jax.numpy as jnp
from jax import lax
from jax.experimental import pallas as pl
from jax.experimental.pallas import tpu as pltpu
from jax.experimental.pallas import tpu_sc as plsc

NC = 2
NS = 16
NW = NC * NS
CHUNK = 80


def _sc_aggregate(xaug, src, dst, zinit, n_pad, da):
    e = src.shape[0]
    epw = e // NW
    nchunk = epw // CHUNK
    rows_per_tile = n_pad // NS

    mesh = plsc.VectorSubcoreMesh(core_axis_name="c", subcore_axis_name="s")

    @functools.partial(
        pl.kernel,
        out_type=jax.ShapeDtypeStruct((NC, n_pad, da), jnp.float32),
        mesh=mesh,
        scratch_types=[
            pltpu.VMEM((CHUNK,), jnp.int32),
            pltpu.VMEM((2, CHUNK), jnp.int32),
            pltpu.VMEM((2, CHUNK, da), jnp.float32),
            pltpu.VMEM_SHARED((n_pad, da), jnp.float32),
            pltpu.SemaphoreType.DMA,
            pltpu.SemaphoreType.DMA((2,)),
        ],
        compiler_params=pltpu.CompilerParams(use_tc_tiling_on_sc=False),
    )
    def agg(xaug_hbm, src_hbm, dst_hbm, zero_hbm, out_hbm,
            sidx_v, didx_v, rows_v, acc_sh, sem, ssem):
        cid = lax.axis_index("c")
        sid = lax.axis_index("s")
        wid = sid * NC + cid
        base = wid * epw
        t0 = sid * rows_per_tile

        pltpu.sync_copy(zero_hbm.at[pl.ds(t0, rows_per_tile)],
                        acc_sh.at[pl.ds(t0, rows_per_tile)])
        plsc.subcore_barrier()

        def wait_scatter(b):
            pltpu.make_async_copy(rows_v.at[b], acc_sh.at[didx_v.at[0]],
                                  ssem.at[b]).wait()

        def body(g, carry):
            for b in range(2):
                j = g * 2 + b

                @pl.when(j < nchunk)
                def _():
                    @pl.when(j >= 2)
                    def _():
                        wait_scatter(b)

                    off = base + j * CHUNK
                    pltpu.sync_copy(src_hbm.at[pl.ds(off, CHUNK)], sidx_v)
                    pltpu.sync_copy(dst_hbm.at[pl.ds(off, CHUNK)],
                                    didx_v.at[b])
                    pltpu.async_copy(xaug_hbm.at[sidx_v], rows_v.at[b],
                                     sem).wait()
                    pltpu.async_copy(rows_v.at[b], acc_sh.at[didx_v.at[b]],
                                     ssem.at[b], add=True)
            return carry

        lax.fori_loop(0, (nchunk + 1) // 2, body, 0)
        wait_scatter(0)
        wait_scatter(1)
        plsc.subcore_barrier()

        pltpu.sync_copy(acc_sh.at[pl.ds(t0, rows_per_tile)],
                        out_hbm.at[cid, pl.ds(t0, rows_per_tile)])

    return agg(xaug, src, dst, zinit)


def _tc_finish_body(p_ref, x_ref, wl_ref, b_ref, wr_ref, o_ref, *, d):
    p = p_ref[...]
    summed = p[0, :, :d] + p[1, :, :d]
    cnt = p[0, :, d] + p[1, :, d]
    mean = summed / jnp.maximum(cnt, 1.0)[:, None]
    o_ref[...] = (
        jnp.dot(mean, wl_ref[...], preferred_element_type=jnp.float32)
        + b_ref[...]
        + jnp.dot(x_ref[...], wr_ref[...], preferred_element_type=jnp.float32)
    )


def kernel(x, edge_index, W_l, b_l, W_r):
    n, d = x.shape
    h = W_l.shape[1]
    e = edge_index.shape[1]
    da = ((d + 1 + 15) // 16) * 16
    n_pad = ((n + 8 * NW - 1) // (8 * NW)) * (8 * NW)
    assert e % (NW * CHUNK) == 0

    xaug = jnp.pad(
        jnp.concatenate([x, jnp.ones((n, 1), x.dtype)], axis=1),
        ((0, n_pad - n), (0, da - d - 1)),
    )
    src = edge_index[0]
    dst = edge_index[1]
    zinit = jnp.zeros((n_pad, da), jnp.float32)

    partial = _sc_aggregate(xaug, src, dst, zinit, n_pad, da)

    blk = 1000
    grid = (n // blk,)
    out = pl.pallas_call(
        functools.partial(_tc_finish_body, d=d),
        grid=grid,
        in_specs=[
            pl.BlockSpec((NC, blk, da), lambda i: (0, i, 0)),
            pl.BlockSpec((blk, d), lambda i: (i, 0)),
            pl.BlockSpec((d, h), lambda i: (0, 0)),
            pl.BlockSpec((1, h), lambda i: (0, 0)),
            pl.BlockSpec((d, h), lambda i: (0, 0)),
        ],
        out_specs=pl.BlockSpec((blk, h), lambda i: (i, 0)),
        out_shape=jax.ShapeDtypeStruct((n, h), jnp.float32),
    )(partial, x, W_l, b_l.reshape(1, h), W_r)
    return out
